# baseline (speedup 1.0000x reference)
import functools

import jax
import jax.numpy as jnp
import numpy as np
from jax import lax
from jax.experimental import pallas as pl
from jax.experimental.pallas import tpu as pltpu
from jax.experimental.pallas import tpu_sc as plsc

_NUM_CLASSES = 80
_C = _NUM_CLASSES + 6
_G = 64
_GG = _G * _G
_NA = 18
_B = 8
_BA = _B * _NA
_PI6 = 0.5235987755982988

_CHUNK = 512
_N_CH = _GG // _CHUNK
_N_WORKERS = 32
_TASKS_PER_W = (_BA * _N_CH) // _N_WORKERS


def _sig(v):
    return 1.0 / (1.0 + jnp.exp(-v))


_mesh = plsc.VectorSubcoreMesh(core_axis_name="c", subcore_axis_name="s")


@functools.partial(
    pl.kernel,
    mesh=_mesh,
    out_type=jax.ShapeDtypeStruct((_BA, _N_CH, _CHUNK * _C), jnp.float32),
    scratch_types=[
        pltpu.VMEM((_C, _CHUNK), jnp.float32),
        pltpu.VMEM((_CHUNK * _C,), jnp.float32),
    ],
    compiler_params=pltpu.CompilerParams(needs_layout_passes=False),
)
def _sc_decode(x_hbm, y_hbm, in_v, out_v):
    wid = lax.axis_index("s") * 2 + lax.axis_index("c")
    lane = lax.iota(jnp.int32, 16)
    lanef = lane.astype(jnp.float32)

    def task(k, carry):
        t = wid + k * _N_WORKERS
        ba = t // _N_CH
        chi = t % _N_CH
        g0 = chi * _CHUNK
        a = ba % _NA
        ai = a // 6
        aj = a % 6
        aw8 = jnp.where(ai == 0, 12.0, jnp.where(ai == 1, 19.0, 40.0))
        ah8 = jnp.where(ai == 0, 16.0, jnp.where(ai == 1, 36.0, 28.0))
        aa = (aj.astype(jnp.float32) - 2.0) * np.float32(_PI6)

        pltpu.sync_copy(x_hbm.at[ba, :, pl.ds(g0, _CHUNK)], in_v)

        def jloop(j, c2):
            gbase = g0 + j * 16
            gxf = (gbase % _G).astype(jnp.float32) + lanef
            gyf = (gbase // _G).astype(jnp.float32)
            gl86 = (j * 16 + lane) * _C
            for c in range(_C):
                v = in_v[c, pl.ds(j * 16, 16)]
                if c == 0:
                    r = _sig(v) * 8.4 + (gxf * 8.0 - 0.2)
                elif c == 1:
                    r = _sig(v) * 8.4 + (gyf * 8.0 - 0.2)
                elif c == 2:
                    r = jnp.exp(v) * aw8
                elif c == 3:
                    r = jnp.exp(v) * ah8
                elif c == 4:
                    r = v + aa
                else:
                    r = _sig(v)
                plsc.store_scatter(out_v, [gl86 + c], r)
            return c2

        lax.fori_loop(0, _CHUNK // 16, jloop, 0)
        pltpu.sync_copy(out_v, y_hbm.at[ba, chi])
        return carry

    lax.fori_loop(0, _TASKS_PER_W, task, 0)


def kernel(output):
    x = output.reshape(_BA, _C, _GG)
    out = _sc_decode(x)
    return out.reshape(_B, _NA * _GG, _C)
